# baseline (speedup 1.0000x reference)
import jax
import jax.numpy as jnp
from jax.experimental import pallas as pl
from jax.experimental.pallas import tpu as pltpu

N = 4096
F2 = 128
G4 = 256

BD = 256
ND = N // BD
BI = 512
NI = N // BI


def _dot(a, b):
    return jax.lax.dot_general(a.astype(jnp.bfloat16), b.astype(jnp.bfloat16),
                               (((1,), (0,)), ((), ())),
                               preferred_element_type=jnp.float32)


def _fused_kernel(l_ref, xht_ref, ct_ref, wt_ref, bcatt_ref,
                  wcit_ref, wcft_ref, wcot_ref,
                  hn_ref, cn_ref,
                  lbft_ref, t1bft_ref, xhbft_ref):
    s = pl.program_id(0)

    @pl.when(s == 0)
    def _():
        xhbft_ref[...] = xht_ref[...].astype(jnp.bfloat16)

    @pl.when(s < ND)
    def _():
        rows = pl.ds(s * BD, BD)
        lbft_ref[:, rows] = l_ref[...].astype(jnp.bfloat16).T

    @pl.when((s >= ND) & (s < ND + NI))
    def _():
        j = s - ND
        colsj = pl.ds(j * BI, BI)
        t1bft_ref[:, colsj] = _dot(xhbft_ref[...],
                                   lbft_ref[:, colsj]).astype(jnp.bfloat16)

    @pl.when(s >= ND + NI)
    def _():
        i = s - ND - NI
        cols = pl.ds(i * BI, BI)
        lt1t = _dot(t1bft_ref[...], lbft_ref[:, cols])
        t0t = xht_ref[:, cols]
        t2t = 2.0 * lt1t - t0t
        wt = wt_ref[...]
        pret = (_dot(wt[0], xhbft_ref[:, cols])
                + _dot(wt[1], t1bft_ref[:, cols])
                + _dot(wt[2], t2t) + bcatt_ref[...])
        cint = ct_ref[:, cols]
        gi = jax.nn.sigmoid(pret[0:64, :] + wcit_ref[...] * cint)
        gf = jax.nn.sigmoid(pret[64:128, :] + wcft_ref[...] * cint)
        gt = jnp.tanh(pret[128:192, :])
        cnt = gf * cint + gi * gt
        go = jax.nn.sigmoid(pret[192:256, :] + wcot_ref[...] * cnt)
        hnt = go * jnp.tanh(cnt)
        hn_ref[...] = hnt.T
        cn_ref[...] = cnt.T


@jax.jit
def _run(XHT, L, CT, WT, bcatT, wciT, wcfT, wcoT):
    hn, cn = pl.pallas_call(
        _fused_kernel,
        grid=(ND + 2 * NI,),
        in_specs=[
            pl.BlockSpec((BD, N), lambda s: (jnp.minimum(s, ND - 1), 0)),
            pl.BlockSpec((F2, N), lambda s: (0, 0)),
            pl.BlockSpec((64, N), lambda s: (0, 0)),
            pl.BlockSpec((3, G4, F2), lambda s: (0, 0, 0)),
            pl.BlockSpec((G4, 1), lambda s: (0, 0)),
            pl.BlockSpec((64, 1), lambda s: (0, 0)),
            pl.BlockSpec((64, 1), lambda s: (0, 0)),
            pl.BlockSpec((64, 1), lambda s: (0, 0)),
        ],
        out_specs=[
            pl.BlockSpec((BI, 64), lambda s: (jnp.maximum(s - ND - NI, 0), 0)),
            pl.BlockSpec((BI, 64), lambda s: (jnp.maximum(s - ND - NI, 0), 0)),
        ],
        out_shape=[
            jax.ShapeDtypeStruct((N, 64), jnp.float32),
            jax.ShapeDtypeStruct((N, 64), jnp.float32),
        ],
        scratch_shapes=[
            pltpu.VMEM((N, N), jnp.bfloat16),
            pltpu.VMEM((F2, N), jnp.bfloat16),
            pltpu.VMEM((F2, N), jnp.bfloat16),
        ],
        compiler_params=pltpu.CompilerParams(
            dimension_semantics=("arbitrary",)),
    )(L, XHT, CT, WT, bcatT, wciT, wcfT, wcoT)
    return hn, cn


def kernel(X, L, H, C,
           W_x_i, b_x_i, W_h_i, b_h_i,
           W_x_f, b_x_f, W_h_f, b_h_f,
           W_x_c, b_x_c, W_h_c, b_h_c,
           W_x_o, b_x_o, W_h_o, b_h_o,
           w_c_i, w_c_f, w_c_o, b_i, b_f, b_c, b_o):
    XHT = jnp.concatenate([X, H], axis=1).T
    Wx = jnp.concatenate([W_x_i, W_x_f, W_x_c, W_x_o], axis=2)
    Wh = jnp.concatenate([W_h_i, W_h_f, W_h_c, W_h_o], axis=2)
    W = jnp.concatenate([Wx, Wh], axis=1)
    WT = jnp.transpose(W, (0, 2, 1))
    bcatT = jnp.concatenate([
        (b_x_i + b_h_i)[None, :] + b_i,
        (b_x_f + b_h_f)[None, :] + b_f,
        (b_x_c + b_h_c)[None, :] + b_c,
        (b_x_o + b_h_o)[None, :] + b_o,
    ], axis=1).T
    return _run(XHT, L, C.T, WT, bcatT, w_c_i.T, w_c_f.T, w_c_o.T)


# speedup vs baseline: 1.2037x; 1.2037x over previous
import jax
import jax.numpy as jnp
from jax.experimental import pallas as pl
from jax.experimental.pallas import tpu as pltpu

N = 4096
F2 = 128
G4 = 256

BI = 512
NI = N // BI


def _dot(a, b):
    return jax.lax.dot_general(a.astype(jnp.bfloat16), b.astype(jnp.bfloat16),
                               (((1,), (0,)), ((), ())),
                               preferred_element_type=jnp.float32)


def _fused_kernel(l_ref, xh_ref, c_ref, w_ref, bcat_ref,
                  wci_ref, wcf_ref, wco_ref,
                  hn_ref, cn_ref,
                  lbf_ref, t1bf_ref, xhbf_ref):
    p = pl.program_id(0)
    i = pl.program_id(1)
    rows = pl.ds(i * BI, BI)

    @pl.when(p == 0)
    def _():
        @pl.when(i == 0)
        def _():
            xhbf_ref[...] = xh_ref[...].astype(jnp.bfloat16)
        lblk = l_ref[...].astype(jnp.bfloat16)
        lbf_ref[rows, :] = lblk
        t1bf_ref[rows, :] = _dot(lblk, xhbf_ref[...]).astype(jnp.bfloat16)

    @pl.when(p == 1)
    def _():
        lt1 = _dot(lbf_ref[rows, :], t1bf_ref[...])
        t0 = xh_ref[rows, :]
        t2 = 2.0 * lt1 - t0
        w = w_ref[...]
        pre = (_dot(xhbf_ref[rows, :], w[0]) + _dot(t1bf_ref[rows, :], w[1])
               + _dot(t2, w[2]) + bcat_ref[...])
        cin = c_ref[rows, :]
        gi = jax.nn.sigmoid(pre[:, 0:64] + wci_ref[...] * cin)
        gf = jax.nn.sigmoid(pre[:, 64:128] + wcf_ref[...] * cin)
        gt = jnp.tanh(pre[:, 128:192])
        cn = gf * cin + gi * gt
        go = jax.nn.sigmoid(pre[:, 192:256] + wco_ref[...] * cn)
        hn_ref[...] = go * jnp.tanh(cn)
        cn_ref[...] = cn


@jax.jit
def _run(XH, L, C, W, bcat, wci, wcf, wco):
    hn, cn = pl.pallas_call(
        _fused_kernel,
        grid=(2, NI),
        in_specs=[
            pl.BlockSpec((BI, N), lambda p, i: (i + p * (NI - 1 - i), 0)),
            pl.BlockSpec((N, F2), lambda p, i: (0, 0)),
            pl.BlockSpec((N, 64), lambda p, i: (0, 0)),
            pl.BlockSpec((3, F2, G4), lambda p, i: (0, 0, 0)),
            pl.BlockSpec((1, G4), lambda p, i: (0, 0)),
            pl.BlockSpec((1, 64), lambda p, i: (0, 0)),
            pl.BlockSpec((1, 64), lambda p, i: (0, 0)),
            pl.BlockSpec((1, 64), lambda p, i: (0, 0)),
        ],
        out_specs=[
            pl.BlockSpec((BI, 64), lambda p, i: (i * p, 0)),
            pl.BlockSpec((BI, 64), lambda p, i: (i * p, 0)),
        ],
        out_shape=[
            jax.ShapeDtypeStruct((N, 64), jnp.float32),
            jax.ShapeDtypeStruct((N, 64), jnp.float32),
        ],
        scratch_shapes=[
            pltpu.VMEM((N, N), jnp.bfloat16),
            pltpu.VMEM((N, F2), jnp.bfloat16),
            pltpu.VMEM((N, F2), jnp.bfloat16),
        ],
        compiler_params=pltpu.CompilerParams(
            dimension_semantics=("arbitrary", "arbitrary")),
    )(L, XH, C, W, bcat, wci, wcf, wco)
    return hn, cn


def kernel(X, L, H, C,
           W_x_i, b_x_i, W_h_i, b_h_i,
           W_x_f, b_x_f, W_h_f, b_h_f,
           W_x_c, b_x_c, W_h_c, b_h_c,
           W_x_o, b_x_o, W_h_o, b_h_o,
           w_c_i, w_c_f, w_c_o, b_i, b_f, b_c, b_o):
    XH = jnp.concatenate([X, H], axis=1)
    Wx = jnp.concatenate([W_x_i, W_x_f, W_x_c, W_x_o], axis=2)
    Wh = jnp.concatenate([W_h_i, W_h_f, W_h_c, W_h_o], axis=2)
    W = jnp.concatenate([Wx, Wh], axis=1)
    bcat = jnp.concatenate([
        (b_x_i + b_h_i)[None, :] + b_i,
        (b_x_f + b_h_f)[None, :] + b_f,
        (b_x_c + b_h_c)[None, :] + b_c,
        (b_x_o + b_h_o)[None, :] + b_o,
    ], axis=1)
    return _run(XH, L, C, W, bcat, w_c_i, w_c_f, w_c_o)
